# baseline (speedup 1.0000x reference)
import jax
import jax.numpy as jnp
from jax.experimental import pallas as pl
from jax.experimental.pallas import tpu as pltpu


def _copy_body(f_in, m_in, f_out, m_out, mask_out, sem_f, sem_m):
    mask_out[...] = jnp.ones(mask_out.shape, dtype=jnp.bool_)
    cf = pltpu.make_async_copy(f_in, f_out, sem_f)
    cm = pltpu.make_async_copy(m_in, m_out, sem_m)
    cf.start()
    cm.start()
    cf.wait()
    cm.wait()


def kernel(features, means, xy_coords, A):
    B, V, G, C = features.shape
    del xy_coords, A
    f2 = features.reshape(B * V * G, C)
    m2 = means.reshape(B * V, G * 3)

    f_out, m_out, mask = pl.pallas_call(
        _copy_body,
        in_specs=[
            pl.BlockSpec(memory_space=pl.ANY),
            pl.BlockSpec(memory_space=pl.ANY),
        ],
        out_specs=[
            pl.BlockSpec(memory_space=pl.ANY),
            pl.BlockSpec(memory_space=pl.ANY),
            pl.BlockSpec(memory_space=pltpu.MemorySpace.VMEM),
        ],
        out_shape=[
            jax.ShapeDtypeStruct((B * V * G, C), features.dtype),
            jax.ShapeDtypeStruct((B * V, G * 3), means.dtype),
            jax.ShapeDtypeStruct((B * V, G), jnp.bool_),
        ],
        scratch_shapes=[pltpu.SemaphoreType.DMA, pltpu.SemaphoreType.DMA],
    )(f2, m2)

    return (
        f_out.reshape(B, V * G, C),
        m_out.reshape(B, V * G, 3),
        mask.reshape(B, V, G),
    )
